# Initial kernel scaffold; baseline (speedup 1.0000x reference)
import jax
import jax.numpy as jnp
from jax.experimental import pallas as pl


def kernel(x, slots_noise, mu, logsigma, Wq, bq, Wk, bk, Wv, bv, W_ih, W_hh, b_ih, b_hh, W1, b1, W2, b2, g_in, b_in, g_slots, b_slots, g_ff, b_ff):
    raise NotImplementedError("write your pallas kernel here")


# speedup vs baseline: 1.0608x; 1.0608x over previous
import functools

import jax
import jax.numpy as jnp
from jax.experimental import pallas as pl
from jax.experimental.pallas import tpu as pltpu

B, N, D = 32, 1024, 768
S = 8
H = 1536
ITERS = 3
EPS = 1e-8


def _ln(x, g, b):
    m = jnp.mean(x, axis=-1, keepdims=True)
    v = jnp.mean((x - m) ** 2, axis=-1, keepdims=True)
    return (x - m) * jax.lax.rsqrt(v + 1e-5) * g + b


def _sa_kernel(x_ref, noise_ref, mu_ref, sigma_ref,
               WqT_ref, bq_ref, WkT_ref, bk_ref, WvT_ref, bv_ref,
               WihT_ref, WhhT_ref, bih_ref, bhh_ref,
               W1T_ref, b1_ref, W2T_ref, b2_ref,
               g_in_ref, b_in_ref, g_s_ref, b_s_ref, g_ff_ref, b_ff_ref,
               out_ref):
    scale = D ** -0.5
    x = x_ref[0]
    xh = _ln(x, g_in_ref[...], b_in_ref[...])
    k = jnp.dot(xh, WkT_ref[...], preferred_element_type=jnp.float32) + bk_ref[...]
    v = jnp.dot(xh, WvT_ref[...], preferred_element_type=jnp.float32) + bv_ref[...]

    slots = mu_ref[0] + sigma_ref[0] * noise_ref[0]

    for _ in range(ITERS):
        slots_prev = slots
        slots_n = _ln(slots, g_s_ref[...], b_s_ref[...])
        q = jnp.dot(slots_n, WqT_ref[...], preferred_element_type=jnp.float32) + bq_ref[...]
        dots = jax.lax.dot_general(
            q, k, (((1,), (1,)), ((), ())),
            preferred_element_type=jnp.float32) * scale
        dmax = jnp.max(dots, axis=0, keepdims=True)
        e = jnp.exp(dots - dmax)
        attn = e / jnp.sum(e, axis=0, keepdims=True) + EPS
        attn = attn / jnp.sum(attn, axis=1, keepdims=True)
        updates = jnp.dot(attn, v, preferred_element_type=jnp.float32)

        gi = jnp.dot(updates, WihT_ref[...], preferred_element_type=jnp.float32) + bih_ref[...]
        gh = jnp.dot(slots_prev, WhhT_ref[...], preferred_element_type=jnp.float32) + bhh_ref[...]
        r = jax.nn.sigmoid(gi[:, :D] + gh[:, :D])
        z = jax.nn.sigmoid(gi[:, D:2 * D] + gh[:, D:2 * D])
        n_ = jnp.tanh(gi[:, 2 * D:] + r * gh[:, 2 * D:])
        slots = (1.0 - z) * n_ + z * slots_prev

        ffx = _ln(slots, g_ff_ref[...], b_ff_ref[...])
        ff = jnp.dot(jax.nn.relu(
            jnp.dot(ffx, W1T_ref[...], preferred_element_type=jnp.float32) + b1_ref[...]),
            W2T_ref[...], preferred_element_type=jnp.float32) + b2_ref[...]
        slots = slots + ff

    out_ref[0] = slots


@jax.jit
def kernel(x, slots_noise, mu, logsigma, Wq, bq, Wk, bk, Wv, bv,
           W_ih, W_hh, b_ih, b_hh, W1, b1, W2, b2,
           g_in, b_in, g_slots, b_slots, g_ff, b_ff):
    row = lambda a: a.reshape(1, -1)
    full = lambda s: pl.BlockSpec(s, lambda b: (0,) * len(s))
    args = (
        x, slots_noise, mu, jnp.exp(logsigma),
        Wq.T, row(bq), Wk.T, row(bk), Wv.T, row(bv),
        W_ih.T, W_hh.T, row(b_ih), row(b_hh),
        W1.T, row(b1), W2.T, row(b2),
        row(g_in), row(b_in), row(g_slots), row(b_slots), row(g_ff), row(b_ff),
    )
    in_specs = [
        pl.BlockSpec((1, N, D), lambda b: (b, 0, 0)),
        pl.BlockSpec((1, S, D), lambda b: (b, 0, 0)),
        full((1, 1, D)), full((1, 1, D)),
        full((D, D)), full((1, D)), full((D, D)), full((1, D)),
        full((D, D)), full((1, D)),
        full((D, 3 * D)), full((D, 3 * D)), full((1, 3 * D)), full((1, 3 * D)),
        full((D, H)), full((1, H)), full((H, D)), full((1, D)),
        full((1, D)), full((1, D)), full((1, D)), full((1, D)),
        full((1, D)), full((1, D)),
    ]
    out = pl.pallas_call(
        _sa_kernel,
        grid=(B,),
        in_specs=in_specs,
        out_specs=pl.BlockSpec((1, S, D), lambda b: (b, 0, 0)),
        out_shape=jax.ShapeDtypeStruct((B, S, D), jnp.float32),
    )(*args)
    return out
